# baseline (speedup 1.0000x reference)
import functools
import jax
import jax.numpy as jnp
import numpy as np
from jax import lax
from jax.experimental import pallas as pl
from jax.experimental.pallas import tpu as pltpu
from jax.experimental.pallas import tpu_sc as plsc


_NC, _NS = 2, 16
_NW = _NC * _NS
_LANES = 16


def _lane_iota():
    return lax.iota(jnp.int32, _LANES)


def _splat_i32(v):
    return jnp.full((_LANES,), v, jnp.int32)


def _make_sc_fps_kernel(N, S, interpret=False):
    mesh = plsc.VectorSubcoreMesh(core_axis_name="c", subcore_axis_name="s",
                                  num_cores=_NC, num_subcores=_NS)
    out_type = [jax.ShapeDtypeStruct((_NW, S), jnp.float32)] * 3
    scratch = [
        pltpu.VMEM((N,), jnp.float32),
        pltpu.VMEM((N,), jnp.float32),
        pltpu.VMEM((N,), jnp.float32),
        pltpu.VMEM((N,), jnp.float32),
        pltpu.VMEM((S,), jnp.float32),
        pltpu.VMEM((S,), jnp.float32),
        pltpu.VMEM((S,), jnp.float32),
    ]

    def body(px_h, py_h, pz_h, nx_h, ny_h, nz_h,
             pxv, pyv, pzv, dist, nxv, nyv, nzv):
        b = lax.axis_index("s") * _NC + lax.axis_index("c")
        pltpu.sync_copy(px_h.at[b], pxv)
        pltpu.sync_copy(py_h.at[b], pyv)
        pltpu.sync_copy(pz_h.at[b], pzv)

        lanes = _lane_iota()
        lane0 = lanes == 0

        def initc(c, _):
            dist[pl.ds(c * 16, 16)] = jnp.full((16,), 1e10, jnp.float32)
            return 0
        lax.fori_loop(0, N // 16, initc, 0)

        def fps_step(i, far):
            cx = plsc.load_gather(pxv, [far])
            cy = plsc.load_gather(pyv, [far])
            cz = plsc.load_gather(pzv, [far])
            isp = _splat_i32(i)
            plsc.store_scatter(nxv, [isp], cx, mask=lane0)
            plsc.store_scatter(nyv, [isp], cy, mask=lane0)
            plsc.store_scatter(nzv, [isp], cz, mask=lane0)

            def chunk(c, carry):
                maxv, argv = carry
                base = c * 16
                dx = pxv[pl.ds(base, 16)] - cx
                dy = pyv[pl.ds(base, 16)] - cy
                dz = pzv[pl.ds(base, 16)] - cz
                d = dx * dx + dy * dy + dz * dz
                nd = jnp.minimum(dist[pl.ds(base, 16)], d)
                dist[pl.ds(base, 16)] = nd
                idxs = base + lanes
                better = nd > maxv
                return (jnp.where(better, nd, maxv),
                        jnp.where(better, idxs, argv))

            maxv, argv = lax.fori_loop(
                0, N // 16, chunk,
                (jnp.full((16,), -1.0, jnp.float32), _splat_i32(0)))
            m = jnp.max(maxv, axis=0)
            cand = maxv == jnp.full((16,), m, jnp.float32)
            argm = jnp.where(cand, argv, _splat_i32(N))
            return _splat_i32(jnp.min(argm, axis=0))

        lax.fori_loop(0, S, fps_step, _splat_i32(0))

        pltpu.sync_copy(nxv, nx_h.at[b])
        pltpu.sync_copy(nyv, ny_h.at[b])
        pltpu.sync_copy(nzv, nz_h.at[b])

    return functools.partial(
        pl.kernel, body, out_type=tuple(out_type), mesh=mesh,
        scratch_types=tuple(scratch), interpret=interpret,
        compiler_params=pltpu.CompilerParams(needs_layout_passes=False))()


def _make_sc_group_kernel(N, S, K, radius, C, interpret=False):
    r2 = jnp.float32(radius * radius)
    mesh = plsc.VectorSubcoreMesh(core_axis_name="c", subcore_axis_name="s",
                                  num_cores=_NC, num_subcores=_NS)
    out_type = [jax.ShapeDtypeStruct((_NW, S * K), jnp.float32)] * 3
    scratch = [
        pltpu.VMEM((N,), jnp.float32),
        pltpu.VMEM((N,), jnp.float32),
        pltpu.VMEM((N,), jnp.float32),
        pltpu.VMEM((S,), jnp.float32),
        pltpu.VMEM((S,), jnp.float32),
        pltpu.VMEM((S,), jnp.float32),
        pltpu.VMEM((16, N), jnp.float32),
        pltpu.VMEM((S * K,), jnp.float32),
        pltpu.VMEM((S * K,), jnp.float32),
        pltpu.VMEM((S * K,), jnp.float32),
    ]
    if C > 0:
        out_type.append(jax.ShapeDtypeStruct((_NW * S * K, C), jnp.float32))
        scratch += [
            pltpu.VMEM((S * K,), jnp.int32),
            pltpu.VMEM((K, C), jnp.float32),
            pltpu.SemaphoreType.DMA,
        ]

    def body(*refs):
        if C > 0:
            (px_h, py_h, pz_h, nx_h, ny_h, nz_h, sq_h, feats_h,
             gx_h, gy_h, gz_h, gf_h,
             pxv, pyv, pzv, nxv, nyv, nzv, dslab, gxv, gyv, gzv,
             idxv, growv, dsem) = refs
        else:
            (px_h, py_h, pz_h, nx_h, ny_h, nz_h, sq_h,
             gx_h, gy_h, gz_h,
             pxv, pyv, pzv, nxv, nyv, nzv, dslab, gxv, gyv, gzv) = refs

        b = lax.axis_index("s") * _NC + lax.axis_index("c")
        pltpu.sync_copy(px_h.at[b], pxv)
        pltpu.sync_copy(py_h.at[b], pyv)
        pltpu.sync_copy(pz_h.at[b], pzv)
        pltpu.sync_copy(nx_h.at[b], nxv)
        pltpu.sync_copy(ny_h.at[b], nyv)
        pltpu.sync_copy(nz_h.at[b], nzv)

        lanes = _lane_iota()

        def row_group(g, _):
            rows = g * 16 + lanes
            pltpu.sync_copy(sq_h.at[b, pl.ds(g * 16, 16)], dslab)
            cx = plsc.load_gather(nxv, [rows])
            cy = plsc.load_gather(nyv, [rows])
            cz = plsc.load_gather(nzv, [rows])
            rowbase = rows * K

            def pt(n, counts):
                nsp = _splat_i32(n)
                d = plsc.load_gather(dslab, [lanes, nsp])
                dx = plsc.load_gather(pxv, [nsp]) - cx
                dy = plsc.load_gather(pyv, [nsp]) - cy
                dz = plsc.load_gather(pzv, [nsp]) - cz
                sel = jnp.logical_and(d <= r2, counts < K)
                pos = rowbase + counts
                plsc.store_scatter(gxv, [pos], dx, mask=sel)
                plsc.store_scatter(gyv, [pos], dy, mask=sel)
                plsc.store_scatter(gzv, [pos], dz, mask=sel)
                if C > 0:
                    plsc.store_scatter(idxv, [pos], _splat_i32(b * N + n),
                                       mask=sel)
                return counts + sel.astype(jnp.int32)

            counts = lax.fori_loop(0, N, pt, _splat_i32(0))

            empty = counts == 0
            lastsp = _splat_i32(N - 1)
            fx = jnp.where(empty, plsc.load_gather(pxv, [lastsp]) - cx,
                           plsc.load_gather(gxv, [rowbase]))
            fy = jnp.where(empty, plsc.load_gather(pyv, [lastsp]) - cy,
                           plsc.load_gather(gyv, [rowbase]))
            fz = jnp.where(empty, plsc.load_gather(pzv, [lastsp]) - cz,
                           plsc.load_gather(gzv, [rowbase]))
            if C > 0:
                fi = jnp.where(empty, _splat_i32(b * N + N - 1),
                               plsc.load_gather(idxv, [rowbase]))

            def fillk(k, _):
                pos = rowbase + _splat_i32(k)
                need = _splat_i32(k) >= counts
                plsc.store_scatter(gxv, [pos], fx, mask=need)
                plsc.store_scatter(gyv, [pos], fy, mask=need)
                plsc.store_scatter(gzv, [pos], fz, mask=need)
                if C > 0:
                    plsc.store_scatter(idxv, [pos], fi, mask=need)
                return 0
            lax.fori_loop(0, K, fillk, 0)
            return 0

        lax.fori_loop(0, S // 16, row_group, 0)

        pltpu.sync_copy(gxv, gx_h.at[b])
        pltpu.sync_copy(gyv, gy_h.at[b])
        pltpu.sync_copy(gzv, gz_h.at[b])

        if C > 0:
            def feat_row(s, _):
                idx_slice = idxv.at[pl.ds(s * K, K)]
                pltpu.async_copy(feats_h.at[idx_slice], growv, dsem).wait()
                pltpu.sync_copy(growv, gf_h.at[pl.ds((b * S + s) * K, K)])
                return 0
            lax.fori_loop(0, S, feat_row, 0)

    return functools.partial(
        pl.kernel, body, out_type=tuple(out_type), mesh=mesh,
        scratch_types=tuple(scratch), interpret=interpret,
        compiler_params=pltpu.CompilerParams(needs_layout_passes=False))()


def _sqdist(src, dst):
    return (jnp.sum(src ** 2, -1)[:, :, None]
            + jnp.sum(dst ** 2, -1)[:, None, :]
            - 2.0 * jnp.einsum('bsc,bnc->bsn', src, dst))


def _gather_rows(points, idx):
    return jax.vmap(lambda p, i: p[i])(points, idx)


def _fps(xyz, npoint):
    B, N, _ = xyz.shape

    def step(carry, _):
        distance, farthest = carry
        centroid = jax.vmap(lambda p, f: p[f])(xyz, farthest)[:, None, :]
        dist = jnp.sum((xyz - centroid) ** 2, -1)
        distance = jnp.minimum(distance, dist)
        new_farthest = jnp.argmax(distance, axis=-1).astype(jnp.int32)
        return (distance, new_farthest), farthest

    init = (jnp.full((B, N), 1e10, jnp.float32), jnp.zeros((B,), jnp.int32))
    _, centroids = jax.lax.scan(step, init, None, length=npoint)
    return jnp.transpose(centroids)


def _ball_query(radius, nsample, xyz, new_xyz):
    B, S, _ = new_xyz.shape
    N = xyz.shape[1]
    sqrdists = _sqdist(new_xyz, xyz)
    group_idx = jnp.broadcast_to(jnp.arange(N, dtype=jnp.int32), (B, S, N))
    group_idx = jnp.where(sqrdists > radius ** 2, N, group_idx)
    group_idx = jnp.sort(group_idx, axis=-1)[:, :, :nsample]
    group_first = jnp.broadcast_to(group_idx[:, :, 0:1], group_idx.shape)
    group_idx = jnp.where(group_idx == N, group_first, group_idx)
    return group_idx


def _conv_bn_relu(x, layer):
    W, b, gamma, beta = layer
    x = jnp.einsum('bskc,cd->bskd', x, W) + b
    mean = jnp.mean(x, axis=(0, 1, 2), keepdims=True)
    var = jnp.var(x, axis=(0, 1, 2), keepdims=True)
    x = (x - mean) / jnp.sqrt(var + 1e-5) * gamma + beta
    return jax.nn.relu(x)


def _sa(xyz, points, npoint, radius, nsample, layers, group_all):
    B = xyz.shape[0]
    if group_all:
        new_xyz = jnp.zeros((B, 1, 3), xyz.dtype)
        new_points = xyz[:, None, :, :]
        if points is not None:
            new_points = jnp.concatenate([new_points, points[:, None, :, :]], -1)
    else:
        fps_idx = _fps(xyz, npoint)
        new_xyz = _gather_rows(xyz, fps_idx)
        idx = _ball_query(radius, nsample, xyz, new_xyz)
        grouped_xyz = _gather_rows(xyz, idx) - new_xyz[:, :, None, :]
        if points is not None:
            new_points = jnp.concatenate([grouped_xyz, _gather_rows(points, idx)], -1)
        else:
            new_points = grouped_xyz
    for layer in layers:
        new_points = _conv_bn_relu(new_points, layer)
    new_points = jnp.max(new_points, axis=2)
    return new_xyz, new_points


def _mlp_pool(new_points, layers):
    for layer in layers:
        new_points = _conv_bn_relu(new_points, layer)
    return jnp.max(new_points, axis=2)


_R = 512


def _dotT(w_ref, x_ref, contract_x):
    return lax.dot_general(
        w_ref[...], x_ref[...], (((0,), (contract_x,)), ((), ())),
        preferred_element_type=jnp.float32)


def _bn_scale_shift(acc, P, gamma, beta):
    C = acc.shape[0] // 2
    mean = jnp.sum(acc[:C], axis=1) / P
    var = jnp.sum(acc[C:], axis=1) / P - mean * mean
    s = gamma / jnp.sqrt(var + 1e-5)
    sh = beta - mean * s
    return s[:, None], sh[:, None]


def _tc_two_layer_stage(xt, gf, layers, K, P):
    (W1, b1, ga1, be1), (W2, b2, ga2, be2) = layers
    Cf = 0 if gf is None else gf.shape[1]
    C1, C2 = W1.shape[1], W2.shape[1]
    G = _R // K
    grid = P // _R
    W1x = jnp.zeros((8, C1), jnp.float32).at[:3].set(W1[:3])
    W1f = W1[3:] if Cf else None

    wspec = lambda shape: pl.BlockSpec(shape, lambda i: (0, 0))
    xt_spec = pl.BlockSpec((8, _R), lambda i: (0, i))
    gf_spec = pl.BlockSpec((_R, Cf), lambda i: (i, 0))
    cparams = pltpu.CompilerParams(dimension_semantics=("arbitrary",))

    def y1_of(xt_ref, gf_ref, w1x_ref, w1f_ref, b1_ref):
        y = _dotT(w1x_ref, xt_ref, 0)
        if Cf:
            y = y + _dotT(w1f_ref, gf_ref, 1)
        return y + b1_ref[...]

    gf_args = ((gf,) if Cf else ())
    gf_specs = ((gf_spec,) if Cf else ())
    w1f_specs = ((wspec((Cf, C1)),) if Cf else ())
    w1f_args = ((W1f,) if Cf else ())

    def ka(*refs):
        if Cf:
            xt_ref, gf_ref, w1x_ref, w1f_ref, b1_ref, acc_ref = refs
        else:
            xt_ref, w1x_ref, b1_ref, acc_ref = refs
            gf_ref = w1f_ref = None
        y = y1_of(xt_ref, gf_ref, w1x_ref, w1f_ref, b1_ref)

        @pl.when(pl.program_id(0) == 0)
        def _():
            acc_ref[...] = jnp.zeros_like(acc_ref)
        acc_ref[0:C1, :] += y
        acc_ref[C1:, :] += y * y

    acc1 = pl.pallas_call(
        ka, grid=(grid,),
        in_specs=[xt_spec, *gf_specs, wspec((8, C1)), *w1f_specs,
                  wspec((C1, 1))],
        out_specs=pl.BlockSpec((2 * C1, _R), lambda i: (0, 0)),
        out_shape=jax.ShapeDtypeStruct((2 * C1, _R), jnp.float32),
        compiler_params=cparams, interpret=_INTERPRET,
    )(xt, *gf_args, W1x, *w1f_args, b1[:, None])
    s1, sh1 = _bn_scale_shift(acc1, P, ga1, be1)

    def kb(*refs):
        if Cf:
            (xt_ref, gf_ref, w1x_ref, w1f_ref, b1_ref, s1_ref, sh1_ref,
             w2_ref, b2_ref, acc_ref) = refs
        else:
            (xt_ref, w1x_ref, b1_ref, s1_ref, sh1_ref,
             w2_ref, b2_ref, acc_ref) = refs
            gf_ref = w1f_ref = None
        y = y1_of(xt_ref, gf_ref, w1x_ref, w1f_ref, b1_ref)
        h1 = jnp.maximum(y * s1_ref[...] + sh1_ref[...], 0.0)
        y2 = _dotT(w2_ref, h1, 0) + b2_ref[...]

        @pl.when(pl.program_id(0) == 0)
        def _():
            acc_ref[...] = jnp.zeros_like(acc_ref)
        acc_ref[0:C2, :] += y2
        acc_ref[C2:, :] += y2 * y2

    acc2 = pl.pallas_call(
        kb, grid=(grid,),
        in_specs=[xt_spec, *gf_specs, wspec((8, C1)), *w1f_specs,
                  wspec((C1, 1)), wspec((C1, 1)), wspec((C1, 1)),
                  wspec((C1, C2)), wspec((C2, 1))],
        out_specs=pl.BlockSpec((2 * C2, _R), lambda i: (0, 0)),
        out_shape=jax.ShapeDtypeStruct((2 * C2, _R), jnp.float32),
        compiler_params=cparams, interpret=_INTERPRET,
    )(xt, *gf_args, W1x, *w1f_args, b1[:, None], s1, sh1, W2, b2[:, None])
    s2, sh2 = _bn_scale_shift(acc2, P, ga2, be2)

    def kc(*refs):
        if Cf:
            (xt_ref, gf_ref, w1x_ref, w1f_ref, b1_ref, s1_ref, sh1_ref,
             w2_ref, b2_ref, s2_ref, sh2_ref, out_ref) = refs
        else:
            (xt_ref, w1x_ref, b1_ref, s1_ref, sh1_ref,
             w2_ref, b2_ref, s2_ref, sh2_ref, out_ref) = refs
            gf_ref = w1f_ref = None
        y = y1_of(xt_ref, gf_ref, w1x_ref, w1f_ref, b1_ref)
        h1 = jnp.maximum(y * s1_ref[...] + sh1_ref[...], 0.0)
        y2 = _dotT(w2_ref, h1, 0) + b2_ref[...]
        h2 = jnp.maximum(y2 * s2_ref[...] + sh2_ref[...], 0.0)
        out_ref[...] = jnp.concatenate(
            [jnp.max(h2[:, g * K:(g + 1) * K], axis=1, keepdims=True)
             for g in range(G)], axis=1)[None]

    pooled = pl.pallas_call(
        kc, grid=(grid,),
        in_specs=[xt_spec, *gf_specs, wspec((8, C1)), *w1f_specs,
                  wspec((C1, 1)), wspec((C1, 1)), wspec((C1, 1)),
                  wspec((C1, C2)), wspec((C2, 1)),
                  wspec((C2, 1)), wspec((C2, 1))],
        out_specs=pl.BlockSpec((1, C2, G), lambda i: (i, 0, 0)),
        out_shape=jax.ShapeDtypeStruct((grid, C2, G), jnp.float32),
        compiler_params=cparams, interpret=_INTERPRET,
    )(xt, *gf_args, W1x, *w1f_args, b1[:, None], s1, sh1, W2, b2[:, None],
      s2, sh2)
    return jnp.transpose(pooled, (0, 2, 1)).reshape(P // K, C2)


def _tc_sa3_stage(xt, feats, layer, K):
    W, b, ga, be = layer
    P = xt.shape[1]
    C2 = W.shape[1]
    G = P // K
    Wx = jnp.zeros((8, C2), jnp.float32).at[:3].set(W[:3])
    Wf = W[3:]
    Cf = Wf.shape[0]

    def body(xt_ref, f_ref, wx_ref, wf_ref, b_ref, ga_ref, be_ref, out_ref):
        y = _dotT(wx_ref, xt_ref, 0) + _dotT(wf_ref, f_ref, 1) + b_ref[...]
        mean = jnp.sum(y, axis=1, keepdims=True) / P
        var = jnp.sum(y * y, axis=1, keepdims=True) / P - mean * mean
        s = ga_ref[...] / jnp.sqrt(var + 1e-5)
        sh = be_ref[...] - mean * s
        h = jnp.maximum(y * s + sh, 0.0)
        out_ref[...] = jnp.concatenate(
            [jnp.max(h[:, g * K:(g + 1) * K], axis=1, keepdims=True)
             for g in range(G)], axis=1)

    return pl.pallas_call(
        body,
        out_shape=jax.ShapeDtypeStruct((C2, G), jnp.float32),
        interpret=_INTERPRET,
    )(xt, feats, Wx, Wf, b[:, None], ga[:, None], be[:, None])


def _pad8_planes(planes, P):
    xt = jnp.zeros((8, P), jnp.float32)
    for i, p in enumerate(planes):
        xt = xt.at[i].set(p.reshape(P))
    return xt


_INTERPRET = False


def kernel(xyz, params):
    B = xyz.shape[0]
    px, py, pz = xyz[:, 0, :], xyz[:, 1, :], xyz[:, 2, :]

    fps1 = _make_sc_fps_kernel(1024, 512, interpret=_INTERPRET)
    nx1, ny1, nz1 = fps1(px, py, pz)
    new_xyz1 = jnp.stack([nx1, ny1, nz1], axis=-1)
    xyz_t = jnp.transpose(xyz, (0, 2, 1))
    sq1 = _sqdist(new_xyz1, xyz_t)
    sc1 = _make_sc_group_kernel(1024, 512, 32, 0.2, 0, interpret=_INTERPRET)
    g1x, g1y, g1z = sc1(px, py, pz, nx1, ny1, nz1, sq1)
    P1 = B * 512 * 32
    xt1 = _pad8_planes([g1x, g1y, g1z], P1)
    l1_points = _tc_two_layer_stage(xt1, None, params['sa1'], 32, P1)

    fps2 = _make_sc_fps_kernel(512, 128, interpret=_INTERPRET)
    nx2, ny2, nz2 = fps2(nx1, ny1, nz1)
    new_xyz2 = jnp.stack([nx2, ny2, nz2], axis=-1)
    sq2 = _sqdist(new_xyz2, new_xyz1)
    sc2 = _make_sc_group_kernel(512, 128, 64, 0.4, 128, interpret=_INTERPRET)
    g2x, g2y, g2z, g2f = sc2(nx1, ny1, nz1, nx2, ny2, nz2, sq2, l1_points)
    P2 = B * 128 * 64
    xt2 = _pad8_planes([g2x, g2y, g2z], P2)
    l2_points = _tc_two_layer_stage(xt2, g2f, params['sa2'], 64, P2)

    xt3 = _pad8_planes([nx2, ny2, nz2], B * 128)
    pooled3 = _tc_sa3_stage(xt3, l2_points, params['sa3'][0], 128)

    x = pooled3.T
    l3 = x[:, :, None]
    return x, l3


# speedup vs baseline: 1.7290x; 1.7290x over previous
import functools
import jax
import jax.numpy as jnp
import numpy as np
from jax import lax
from jax.experimental import pallas as pl
from jax.experimental.pallas import tpu as pltpu
from jax.experimental.pallas import tpu_sc as plsc


_NC, _NS = 2, 16
_NW = _NC * _NS
_LANES = 16


def _lane_iota():
    return lax.iota(jnp.int32, _LANES)


def _splat_i32(v):
    return jnp.full((_LANES,), v, jnp.int32)


def _make_sc_fps_kernel(N, S, interpret=False):
    mesh = plsc.VectorSubcoreMesh(core_axis_name="c", subcore_axis_name="s",
                                  num_cores=_NC, num_subcores=_NS)
    out_type = [jax.ShapeDtypeStruct((_NW, S), jnp.float32)] * 3
    scratch = [
        pltpu.VMEM((N,), jnp.float32),
        pltpu.VMEM((N,), jnp.float32),
        pltpu.VMEM((N,), jnp.float32),
        pltpu.VMEM((N,), jnp.float32),
        pltpu.VMEM((S,), jnp.float32),
        pltpu.VMEM((S,), jnp.float32),
        pltpu.VMEM((S,), jnp.float32),
    ]

    def body(px_h, py_h, pz_h, nx_h, ny_h, nz_h,
             pxv, pyv, pzv, dist, nxv, nyv, nzv):
        b = lax.axis_index("s") * _NC + lax.axis_index("c")
        pltpu.sync_copy(px_h.at[b], pxv)
        pltpu.sync_copy(py_h.at[b], pyv)
        pltpu.sync_copy(pz_h.at[b], pzv)

        lanes = _lane_iota()
        lane0 = lanes == 0

        def initc(c, _):
            dist[pl.ds(c * 16, 16)] = jnp.full((16,), 1e10, jnp.float32)
            return 0
        lax.fori_loop(0, N // 16, initc, 0)

        def fps_step(i, far):
            cx = plsc.load_gather(pxv, [far])
            cy = plsc.load_gather(pyv, [far])
            cz = plsc.load_gather(pzv, [far])
            isp = _splat_i32(i)
            plsc.store_scatter(nxv, [isp], cx, mask=lane0)
            plsc.store_scatter(nyv, [isp], cy, mask=lane0)
            plsc.store_scatter(nzv, [isp], cz, mask=lane0)

            def chunk(c, carry):
                maxv, argv = carry
                base = c * 16
                dx = pxv[pl.ds(base, 16)] - cx
                dy = pyv[pl.ds(base, 16)] - cy
                dz = pzv[pl.ds(base, 16)] - cz
                d = dx * dx + dy * dy + dz * dz
                nd = jnp.minimum(dist[pl.ds(base, 16)], d)
                dist[pl.ds(base, 16)] = nd
                idxs = base + lanes
                better = nd > maxv
                return (jnp.where(better, nd, maxv),
                        jnp.where(better, idxs, argv))

            maxv, argv = lax.fori_loop(
                0, N // 16, chunk,
                (jnp.full((16,), -1.0, jnp.float32), _splat_i32(0)))
            m = jnp.max(maxv, axis=0)
            cand = maxv == jnp.full((16,), m, jnp.float32)
            argm = jnp.where(cand, argv, _splat_i32(N))
            return _splat_i32(jnp.min(argm, axis=0))

        lax.fori_loop(0, S, fps_step, _splat_i32(0))

        pltpu.sync_copy(nxv, nx_h.at[b])
        pltpu.sync_copy(nyv, ny_h.at[b])
        pltpu.sync_copy(nzv, nz_h.at[b])

    return functools.partial(
        pl.kernel, body, out_type=tuple(out_type), mesh=mesh,
        scratch_types=tuple(scratch), interpret=interpret,
        compiler_params=pltpu.CompilerParams(needs_layout_passes=False))()


def _make_sc_group_kernel(N, S, K, radius, C, interpret=False):
    r2 = jnp.float32(radius * radius)
    mesh = plsc.VectorSubcoreMesh(core_axis_name="c", subcore_axis_name="s",
                                  num_cores=_NC, num_subcores=_NS)
    out_type = [jax.ShapeDtypeStruct((_NW, S * K), jnp.float32)] * 3
    scratch = [
        pltpu.VMEM((N,), jnp.float32),
        pltpu.VMEM((N,), jnp.float32),
        pltpu.VMEM((N,), jnp.float32),
        pltpu.VMEM((S,), jnp.float32),
        pltpu.VMEM((S,), jnp.float32),
        pltpu.VMEM((S,), jnp.float32),
        pltpu.VMEM((16, N), jnp.float32),
        pltpu.VMEM((S * K,), jnp.float32),
        pltpu.VMEM((S * K,), jnp.float32),
        pltpu.VMEM((S * K,), jnp.float32),
    ]
    if C > 0:
        out_type.append(jax.ShapeDtypeStruct((_NW * S * K, C), jnp.float32))
        scratch += [
            pltpu.VMEM((S * K,), jnp.int32),
            pltpu.VMEM((K, C), jnp.float32),
            pltpu.SemaphoreType.DMA,
        ]

    def body(*refs):
        if C > 0:
            (px_h, py_h, pz_h, nx_h, ny_h, nz_h, sq_h, feats_h,
             gx_h, gy_h, gz_h, gf_h,
             pxv, pyv, pzv, nxv, nyv, nzv, dslab, gxv, gyv, gzv,
             idxv, growv, dsem) = refs
        else:
            (px_h, py_h, pz_h, nx_h, ny_h, nz_h, sq_h,
             gx_h, gy_h, gz_h,
             pxv, pyv, pzv, nxv, nyv, nzv, dslab, gxv, gyv, gzv) = refs

        b = lax.axis_index("s") * _NC + lax.axis_index("c")
        pltpu.sync_copy(px_h.at[b], pxv)
        pltpu.sync_copy(py_h.at[b], pyv)
        pltpu.sync_copy(pz_h.at[b], pzv)
        pltpu.sync_copy(nx_h.at[b], nxv)
        pltpu.sync_copy(ny_h.at[b], nyv)
        pltpu.sync_copy(nz_h.at[b], nzv)

        lanes = _lane_iota()

        def row_group(g, _):
            rows = g * 16 + lanes
            pltpu.sync_copy(sq_h.at[b, pl.ds(g * 16, 16)], dslab)
            cx = plsc.load_gather(nxv, [rows])
            cy = plsc.load_gather(nyv, [rows])
            cz = plsc.load_gather(nzv, [rows])
            rowbase = rows * K

            def pt(n, counts):
                nsp = _splat_i32(n)
                d = plsc.load_gather(dslab, [lanes, nsp])
                dx = plsc.load_gather(pxv, [nsp]) - cx
                dy = plsc.load_gather(pyv, [nsp]) - cy
                dz = plsc.load_gather(pzv, [nsp]) - cz
                sel = jnp.logical_and(d <= r2, counts < K)
                pos = rowbase + counts
                plsc.store_scatter(gxv, [pos], dx, mask=sel)
                plsc.store_scatter(gyv, [pos], dy, mask=sel)
                plsc.store_scatter(gzv, [pos], dz, mask=sel)
                if C > 0:
                    plsc.store_scatter(idxv, [pos], _splat_i32(b * N + n),
                                       mask=sel)
                return counts + sel.astype(jnp.int32)

            counts = lax.fori_loop(0, N, pt, _splat_i32(0))

            empty = counts == 0
            lastsp = _splat_i32(N - 1)
            fx = jnp.where(empty, plsc.load_gather(pxv, [lastsp]) - cx,
                           plsc.load_gather(gxv, [rowbase]))
            fy = jnp.where(empty, plsc.load_gather(pyv, [lastsp]) - cy,
                           plsc.load_gather(gyv, [rowbase]))
            fz = jnp.where(empty, plsc.load_gather(pzv, [lastsp]) - cz,
                           plsc.load_gather(gzv, [rowbase]))
            if C > 0:
                fi = jnp.where(empty, _splat_i32(b * N + N - 1),
                               plsc.load_gather(idxv, [rowbase]))

            def fillk(k, _):
                pos = rowbase + _splat_i32(k)
                need = _splat_i32(k) >= counts
                plsc.store_scatter(gxv, [pos], fx, mask=need)
                plsc.store_scatter(gyv, [pos], fy, mask=need)
                plsc.store_scatter(gzv, [pos], fz, mask=need)
                if C > 0:
                    plsc.store_scatter(idxv, [pos], fi, mask=need)
                return 0
            lax.fori_loop(0, K, fillk, 0)
            return 0

        lax.fori_loop(0, S // 16, row_group, 0)

        pltpu.sync_copy(gxv, gx_h.at[b])
        pltpu.sync_copy(gyv, gy_h.at[b])
        pltpu.sync_copy(gzv, gz_h.at[b])

        if C > 0:
            def feat_row(s, _):
                idx_slice = idxv.at[pl.ds(s * K, K)]
                pltpu.async_copy(feats_h.at[idx_slice], growv, dsem).wait()
                pltpu.sync_copy(growv, gf_h.at[pl.ds((b * S + s) * K, K)])
                return 0
            lax.fori_loop(0, S, feat_row, 0)

    return functools.partial(
        pl.kernel, body, out_type=tuple(out_type), mesh=mesh,
        scratch_types=tuple(scratch), interpret=interpret,
        compiler_params=pltpu.CompilerParams(needs_layout_passes=False))()


def _sqdist(src, dst):
    return (jnp.sum(src ** 2, -1)[:, :, None]
            + jnp.sum(dst ** 2, -1)[:, None, :]
            - 2.0 * jnp.einsum('bsc,bnc->bsn', src, dst))


def _gather_rows(points, idx):
    return jax.vmap(lambda p, i: p[i])(points, idx)


def _fps(xyz, npoint):
    B, N, _ = xyz.shape

    def step(carry, _):
        distance, farthest = carry
        centroid = jax.vmap(lambda p, f: p[f])(xyz, farthest)[:, None, :]
        dist = jnp.sum((xyz - centroid) ** 2, -1)
        distance = jnp.minimum(distance, dist)
        new_farthest = jnp.argmax(distance, axis=-1).astype(jnp.int32)
        return (distance, new_farthest), farthest

    init = (jnp.full((B, N), 1e10, jnp.float32), jnp.zeros((B,), jnp.int32))
    _, centroids = jax.lax.scan(step, init, None, length=npoint)
    return jnp.transpose(centroids)


def _ball_query(radius, nsample, xyz, new_xyz):
    B, S, _ = new_xyz.shape
    N = xyz.shape[1]
    sqrdists = _sqdist(new_xyz, xyz)
    group_idx = jnp.broadcast_to(jnp.arange(N, dtype=jnp.int32), (B, S, N))
    group_idx = jnp.where(sqrdists > radius ** 2, N, group_idx)
    group_idx = jnp.sort(group_idx, axis=-1)[:, :, :nsample]
    group_first = jnp.broadcast_to(group_idx[:, :, 0:1], group_idx.shape)
    group_idx = jnp.where(group_idx == N, group_first, group_idx)
    return group_idx


def _conv_bn_relu(x, layer):
    W, b, gamma, beta = layer
    x = jnp.einsum('bskc,cd->bskd', x, W) + b
    mean = jnp.mean(x, axis=(0, 1, 2), keepdims=True)
    var = jnp.var(x, axis=(0, 1, 2), keepdims=True)
    x = (x - mean) / jnp.sqrt(var + 1e-5) * gamma + beta
    return jax.nn.relu(x)


def _sa(xyz, points, npoint, radius, nsample, layers, group_all):
    B = xyz.shape[0]
    if group_all:
        new_xyz = jnp.zeros((B, 1, 3), xyz.dtype)
        new_points = xyz[:, None, :, :]
        if points is not None:
            new_points = jnp.concatenate([new_points, points[:, None, :, :]], -1)
    else:
        fps_idx = _fps(xyz, npoint)
        new_xyz = _gather_rows(xyz, fps_idx)
        idx = _ball_query(radius, nsample, xyz, new_xyz)
        grouped_xyz = _gather_rows(xyz, idx) - new_xyz[:, :, None, :]
        if points is not None:
            new_points = jnp.concatenate([grouped_xyz, _gather_rows(points, idx)], -1)
        else:
            new_points = grouped_xyz
    for layer in layers:
        new_points = _conv_bn_relu(new_points, layer)
    new_points = jnp.max(new_points, axis=2)
    return new_xyz, new_points


def _mlp_pool(new_points, layers):
    for layer in layers:
        new_points = _conv_bn_relu(new_points, layer)
    return jnp.max(new_points, axis=2)


_R = 4096


def _dotT(w_ref, x_ref, contract_x):
    return lax.dot_general(
        w_ref[...], x_ref[...], (((0,), (contract_x,)), ((), ())),
        preferred_element_type=jnp.float32)


def _bn_scale_shift(acc, P, gamma, beta):
    C = acc.shape[0] // 2
    mean = jnp.sum(acc[:C], axis=1) / P
    var = jnp.sum(acc[C:], axis=1) / P - mean * mean
    s = gamma / jnp.sqrt(var + 1e-5)
    sh = beta - mean * s
    return s[:, None], sh[:, None]


def _tc_two_layer_stage(xt, gf, layers, K, P):
    (W1, b1, ga1, be1), (W2, b2, ga2, be2) = layers
    Cf = 0 if gf is None else gf.shape[1]
    C1, C2 = W1.shape[1], W2.shape[1]
    _Rl = min(_R, P)
    G = _Rl // K
    grid = P // _Rl
    W1x = jnp.zeros((8, C1), jnp.float32).at[:3].set(W1[:3])
    W1f = W1[3:] if Cf else None

    wspec = lambda shape: pl.BlockSpec(shape, lambda i: (0, 0))
    xt_spec = pl.BlockSpec((8, _Rl), lambda i: (0, i))
    gf_spec = pl.BlockSpec((_Rl, Cf), lambda i: (i, 0))
    cparams = pltpu.CompilerParams(dimension_semantics=("arbitrary",))

    def y1_of(xt_ref, gf_ref, w1x_ref, w1f_ref, b1_ref):
        y = _dotT(w1x_ref, xt_ref, 0)
        if Cf:
            y = y + _dotT(w1f_ref, gf_ref, 1)
        return y + b1_ref[...]

    gf_args = ((gf,) if Cf else ())
    gf_specs = ((gf_spec,) if Cf else ())
    w1f_specs = ((wspec((Cf, C1)),) if Cf else ())
    w1f_args = ((W1f,) if Cf else ())

    def ka(*refs):
        if Cf:
            xt_ref, gf_ref, w1x_ref, w1f_ref, b1_ref, acc_ref = refs
        else:
            xt_ref, w1x_ref, b1_ref, acc_ref = refs
            gf_ref = w1f_ref = None
        y = y1_of(xt_ref, gf_ref, w1x_ref, w1f_ref, b1_ref)

        @pl.when(pl.program_id(0) == 0)
        def _():
            acc_ref[...] = jnp.zeros_like(acc_ref)
        acc_ref[0:C1, :] += y
        acc_ref[C1:, :] += y * y

    acc1 = pl.pallas_call(
        ka, grid=(grid,),
        in_specs=[xt_spec, *gf_specs, wspec((8, C1)), *w1f_specs,
                  wspec((C1, 1))],
        out_specs=pl.BlockSpec((2 * C1, _Rl), lambda i: (0, 0)),
        out_shape=jax.ShapeDtypeStruct((2 * C1, _Rl), jnp.float32),
        compiler_params=cparams, interpret=_INTERPRET,
    )(xt, *gf_args, W1x, *w1f_args, b1[:, None])
    s1, sh1 = _bn_scale_shift(acc1, P, ga1, be1)

    def kb(*refs):
        if Cf:
            (xt_ref, gf_ref, w1x_ref, w1f_ref, b1_ref, s1_ref, sh1_ref,
             w2_ref, b2_ref, acc_ref) = refs
        else:
            (xt_ref, w1x_ref, b1_ref, s1_ref, sh1_ref,
             w2_ref, b2_ref, acc_ref) = refs
            gf_ref = w1f_ref = None
        y = y1_of(xt_ref, gf_ref, w1x_ref, w1f_ref, b1_ref)
        h1 = jnp.maximum(y * s1_ref[...] + sh1_ref[...], 0.0)
        y2 = _dotT(w2_ref, h1, 0) + b2_ref[...]

        @pl.when(pl.program_id(0) == 0)
        def _():
            acc_ref[...] = jnp.zeros_like(acc_ref)
        acc_ref[0:C2, :] += y2
        acc_ref[C2:, :] += y2 * y2

    acc2 = pl.pallas_call(
        kb, grid=(grid,),
        in_specs=[xt_spec, *gf_specs, wspec((8, C1)), *w1f_specs,
                  wspec((C1, 1)), wspec((C1, 1)), wspec((C1, 1)),
                  wspec((C1, C2)), wspec((C2, 1))],
        out_specs=pl.BlockSpec((2 * C2, _Rl), lambda i: (0, 0)),
        out_shape=jax.ShapeDtypeStruct((2 * C2, _Rl), jnp.float32),
        compiler_params=cparams, interpret=_INTERPRET,
    )(xt, *gf_args, W1x, *w1f_args, b1[:, None], s1, sh1, W2, b2[:, None])
    s2, sh2 = _bn_scale_shift(acc2, P, ga2, be2)

    def kc(*refs):
        if Cf:
            (xt_ref, gf_ref, w1x_ref, w1f_ref, b1_ref, s1_ref, sh1_ref,
             w2_ref, b2_ref, s2_ref, sh2_ref, out_ref) = refs
        else:
            (xt_ref, w1x_ref, b1_ref, s1_ref, sh1_ref,
             w2_ref, b2_ref, s2_ref, sh2_ref, out_ref) = refs
            gf_ref = w1f_ref = None
        y = y1_of(xt_ref, gf_ref, w1x_ref, w1f_ref, b1_ref)
        h1 = jnp.maximum(y * s1_ref[...] + sh1_ref[...], 0.0)
        y2 = _dotT(w2_ref, h1, 0) + b2_ref[...]
        h2 = jnp.maximum(y2 * s2_ref[...] + sh2_ref[...], 0.0)
        out_ref[...] = jnp.concatenate(
            [jnp.max(h2[:, g * K:(g + 1) * K], axis=1, keepdims=True)
             for g in range(G)], axis=1)[None]

    pooled = pl.pallas_call(
        kc, grid=(grid,),
        in_specs=[xt_spec, *gf_specs, wspec((8, C1)), *w1f_specs,
                  wspec((C1, 1)), wspec((C1, 1)), wspec((C1, 1)),
                  wspec((C1, C2)), wspec((C2, 1)),
                  wspec((C2, 1)), wspec((C2, 1))],
        out_specs=pl.BlockSpec((1, C2, G), lambda i: (i, 0, 0)),
        out_shape=jax.ShapeDtypeStruct((grid, C2, G), jnp.float32),
        compiler_params=cparams, interpret=_INTERPRET,
    )(xt, *gf_args, W1x, *w1f_args, b1[:, None], s1, sh1, W2, b2[:, None],
      s2, sh2)
    return jnp.transpose(pooled, (0, 2, 1)).reshape(P // K, C2)


def _tc_sa3_stage(xt, feats, layer, K):
    W, b, ga, be = layer
    P = xt.shape[1]
    C2 = W.shape[1]
    G = P // K
    Wx = jnp.zeros((8, C2), jnp.float32).at[:3].set(W[:3])
    Wf = W[3:]
    Cf = Wf.shape[0]

    def body(xt_ref, f_ref, wx_ref, wf_ref, b_ref, ga_ref, be_ref, out_ref):
        y = _dotT(wx_ref, xt_ref, 0) + _dotT(wf_ref, f_ref, 1) + b_ref[...]
        mean = jnp.sum(y, axis=1, keepdims=True) / P
        var = jnp.sum(y * y, axis=1, keepdims=True) / P - mean * mean
        s = ga_ref[...] / jnp.sqrt(var + 1e-5)
        sh = be_ref[...] - mean * s
        h = jnp.maximum(y * s + sh, 0.0)
        out_ref[...] = jnp.concatenate(
            [jnp.max(h[:, g * K:(g + 1) * K], axis=1, keepdims=True)
             for g in range(G)], axis=1)

    return pl.pallas_call(
        body,
        out_shape=jax.ShapeDtypeStruct((C2, G), jnp.float32),
        interpret=_INTERPRET,
    )(xt, feats, Wx, Wf, b[:, None], ga[:, None], be[:, None])


def _pad8_planes(planes, P):
    xt = jnp.zeros((8, P), jnp.float32)
    for i, p in enumerate(planes):
        xt = xt.at[i].set(p.reshape(P))
    return xt


_INTERPRET = False


def kernel(xyz, params):
    B = xyz.shape[0]
    px, py, pz = xyz[:, 0, :], xyz[:, 1, :], xyz[:, 2, :]

    fps1 = _make_sc_fps_kernel(1024, 512, interpret=_INTERPRET)
    nx1, ny1, nz1 = fps1(px, py, pz)
    new_xyz1 = jnp.stack([nx1, ny1, nz1], axis=-1)
    xyz_t = jnp.transpose(xyz, (0, 2, 1))
    sq1 = _sqdist(new_xyz1, xyz_t)
    sc1 = _make_sc_group_kernel(1024, 512, 32, 0.2, 0, interpret=_INTERPRET)
    g1x, g1y, g1z = sc1(px, py, pz, nx1, ny1, nz1, sq1)
    P1 = B * 512 * 32
    xt1 = _pad8_planes([g1x, g1y, g1z], P1)
    l1_points = _tc_two_layer_stage(xt1, None, params['sa1'], 32, P1)

    fps2 = _make_sc_fps_kernel(512, 128, interpret=_INTERPRET)
    nx2, ny2, nz2 = fps2(nx1, ny1, nz1)
    new_xyz2 = jnp.stack([nx2, ny2, nz2], axis=-1)
    sq2 = _sqdist(new_xyz2, new_xyz1)
    sc2 = _make_sc_group_kernel(512, 128, 64, 0.4, 128, interpret=_INTERPRET)
    g2x, g2y, g2z, g2f = sc2(nx1, ny1, nz1, nx2, ny2, nz2, sq2, l1_points)
    P2 = B * 128 * 64
    xt2 = _pad8_planes([g2x, g2y, g2z], P2)
    l2_points = _tc_two_layer_stage(xt2, g2f, params['sa2'], 64, P2)

    xt3 = _pad8_planes([nx2, ny2, nz2], B * 128)
    pooled3 = _tc_sa3_stage(xt3, l2_points, params['sa3'][0], 128)

    x = pooled3.T
    l3 = x[:, :, None]
    return x, l3


# speedup vs baseline: 1.9173x; 1.1090x over previous
import functools
import jax
import jax.numpy as jnp
import numpy as np
from jax import lax
from jax.experimental import pallas as pl
from jax.experimental.pallas import tpu as pltpu
from jax.experimental.pallas import tpu_sc as plsc


_NC, _NS = 2, 16
_NW = _NC * _NS
_LANES = 16


def _lane_iota():
    return lax.iota(jnp.int32, _LANES)


def _splat_i32(v):
    return jnp.full((_LANES,), v, jnp.int32)


def _make_sc_fps_kernel(N, S, interpret=False):
    mesh = plsc.VectorSubcoreMesh(core_axis_name="c", subcore_axis_name="s",
                                  num_cores=_NC, num_subcores=_NS)
    out_type = [jax.ShapeDtypeStruct((_NW, S), jnp.float32)] * 3
    scratch = [
        pltpu.VMEM((N,), jnp.float32),
        pltpu.VMEM((N,), jnp.float32),
        pltpu.VMEM((N,), jnp.float32),
        pltpu.VMEM((N,), jnp.float32),
        pltpu.VMEM((S,), jnp.float32),
        pltpu.VMEM((S,), jnp.float32),
        pltpu.VMEM((S,), jnp.float32),
    ]

    def body(px_h, py_h, pz_h, nx_h, ny_h, nz_h,
             pxv, pyv, pzv, dist, nxv, nyv, nzv):
        b = lax.axis_index("s") * _NC + lax.axis_index("c")
        pltpu.sync_copy(px_h.at[b], pxv)
        pltpu.sync_copy(py_h.at[b], pyv)
        pltpu.sync_copy(pz_h.at[b], pzv)

        lanes = _lane_iota()
        lane0 = lanes == 0

        def initc(c, _):
            dist[pl.ds(c * 16, 16)] = jnp.full((16,), 1e10, jnp.float32)
            return 0
        lax.fori_loop(0, N // 16, initc, 0)

        def fps_step(i, far):
            cx = plsc.load_gather(pxv, [far])
            cy = plsc.load_gather(pyv, [far])
            cz = plsc.load_gather(pzv, [far])
            isp = _splat_i32(i)
            plsc.store_scatter(nxv, [isp], cx, mask=lane0)
            plsc.store_scatter(nyv, [isp], cy, mask=lane0)
            plsc.store_scatter(nzv, [isp], cz, mask=lane0)

            def chunk(c, carry):
                maxv, argv = carry
                base = c * 16
                dx = pxv[pl.ds(base, 16)] - cx
                dy = pyv[pl.ds(base, 16)] - cy
                dz = pzv[pl.ds(base, 16)] - cz
                d = dx * dx + dy * dy + dz * dz
                nd = jnp.minimum(dist[pl.ds(base, 16)], d)
                dist[pl.ds(base, 16)] = nd
                idxs = base + lanes
                better = nd > maxv
                return (jnp.where(better, nd, maxv),
                        jnp.where(better, idxs, argv))

            def chunk4(c4, carry):
                for j in range(4):
                    carry = chunk(c4 * 4 + j, carry)
                return carry

            maxv, argv = lax.fori_loop(
                0, N // 64, chunk4,
                (jnp.full((16,), -1.0, jnp.float32), _splat_i32(0)))
            m = jnp.max(maxv, axis=0)
            cand = maxv == jnp.full((16,), m, jnp.float32)
            argm = jnp.where(cand, argv, _splat_i32(N))
            return _splat_i32(jnp.min(argm, axis=0))

        lax.fori_loop(0, S, fps_step, _splat_i32(0))

        pltpu.sync_copy(nxv, nx_h.at[b])
        pltpu.sync_copy(nyv, ny_h.at[b])
        pltpu.sync_copy(nzv, nz_h.at[b])

    return functools.partial(
        pl.kernel, body, out_type=tuple(out_type), mesh=mesh,
        scratch_types=tuple(scratch), interpret=interpret,
        compiler_params=pltpu.CompilerParams(needs_layout_passes=False))()


def _make_sc_group_kernel(N, S, K, radius, C, interpret=False):
    r2 = jnp.float32(radius * radius)
    mesh = plsc.VectorSubcoreMesh(core_axis_name="c", subcore_axis_name="s",
                                  num_cores=_NC, num_subcores=_NS)
    out_type = [jax.ShapeDtypeStruct((_NW, S * K), jnp.float32)] * 3
    scratch = [
        pltpu.VMEM((N,), jnp.float32),
        pltpu.VMEM((N,), jnp.float32),
        pltpu.VMEM((N,), jnp.float32),
        pltpu.VMEM((S,), jnp.float32),
        pltpu.VMEM((S,), jnp.float32),
        pltpu.VMEM((S,), jnp.float32),
        pltpu.VMEM((16, N), jnp.float32),
        pltpu.VMEM((S * K,), jnp.float32),
        pltpu.VMEM((S * K,), jnp.float32),
        pltpu.VMEM((S * K,), jnp.float32),
    ]
    if C > 0:
        out_type.append(jax.ShapeDtypeStruct((_NW * S * K, C), jnp.float32))
        scratch += [
            pltpu.VMEM((S * K,), jnp.int32),
            pltpu.VMEM((2 * K, C), jnp.float32),
            pltpu.VMEM((2 * K, C), jnp.float32),
            pltpu.SemaphoreType.DMA,
            pltpu.SemaphoreType.DMA,
        ]

    def body(*refs):
        if C > 0:
            (px_h, py_h, pz_h, nx_h, ny_h, nz_h, sq_h, feats_h,
             gx_h, gy_h, gz_h, gf_h,
             pxv, pyv, pzv, nxv, nyv, nzv, dslab, gxv, gyv, gzv,
             idxv, growv0, growv1, gsem0, gsem1) = refs
        else:
            (px_h, py_h, pz_h, nx_h, ny_h, nz_h, sq_h,
             gx_h, gy_h, gz_h,
             pxv, pyv, pzv, nxv, nyv, nzv, dslab, gxv, gyv, gzv) = refs

        b = lax.axis_index("s") * _NC + lax.axis_index("c")
        pltpu.sync_copy(px_h.at[b], pxv)
        pltpu.sync_copy(py_h.at[b], pyv)
        pltpu.sync_copy(pz_h.at[b], pzv)
        pltpu.sync_copy(nx_h.at[b], nxv)
        pltpu.sync_copy(ny_h.at[b], nyv)
        pltpu.sync_copy(nz_h.at[b], nzv)

        lanes = _lane_iota()

        def row_group(g, _):
            rows = g * 16 + lanes
            pltpu.sync_copy(sq_h.at[b, pl.ds(g * 16, 16)], dslab)
            cx = plsc.load_gather(nxv, [rows])
            cy = plsc.load_gather(nyv, [rows])
            cz = plsc.load_gather(nzv, [rows])
            rowbase = rows * K

            def pt(n, counts):
                nsp = _splat_i32(n)
                d = plsc.load_gather(dslab, [lanes, nsp])
                dx = plsc.load_gather(pxv, [nsp]) - cx
                dy = plsc.load_gather(pyv, [nsp]) - cy
                dz = plsc.load_gather(pzv, [nsp]) - cz
                sel = jnp.logical_and(d <= r2, counts < K)
                pos = rowbase + counts
                plsc.store_scatter(gxv, [pos], dx, mask=sel)
                plsc.store_scatter(gyv, [pos], dy, mask=sel)
                plsc.store_scatter(gzv, [pos], dz, mask=sel)
                if C > 0:
                    plsc.store_scatter(idxv, [pos], _splat_i32(b * N + n),
                                       mask=sel)
                return counts + sel.astype(jnp.int32)

            def pt4(n4, counts):
                for j in range(4):
                    counts = pt(n4 * 4 + j, counts)
                return counts

            counts = lax.fori_loop(0, N // 4, pt4, _splat_i32(0))

            empty = counts == 0
            lastsp = _splat_i32(N - 1)
            fx = jnp.where(empty, plsc.load_gather(pxv, [lastsp]) - cx,
                           plsc.load_gather(gxv, [rowbase]))
            fy = jnp.where(empty, plsc.load_gather(pyv, [lastsp]) - cy,
                           plsc.load_gather(gyv, [rowbase]))
            fz = jnp.where(empty, plsc.load_gather(pzv, [lastsp]) - cz,
                           plsc.load_gather(gzv, [rowbase]))
            if C > 0:
                fi = jnp.where(empty, _splat_i32(b * N + N - 1),
                               plsc.load_gather(idxv, [rowbase]))

            def fillk(k, _):
                pos = rowbase + _splat_i32(k)
                need = _splat_i32(k) >= counts
                plsc.store_scatter(gxv, [pos], fx, mask=need)
                plsc.store_scatter(gyv, [pos], fy, mask=need)
                plsc.store_scatter(gzv, [pos], fz, mask=need)
                if C > 0:
                    plsc.store_scatter(idxv, [pos], fi, mask=need)
                return 0
            lax.fori_loop(0, K, fillk, 0)
            return 0

        lax.fori_loop(0, S // 16, row_group, 0)

        pltpu.sync_copy(gxv, gx_h.at[b])
        pltpu.sync_copy(gyv, gy_h.at[b])
        pltpu.sync_copy(gzv, gz_h.at[b])

        if C > 0:
            SP = S // 2

            def gsrc(p):
                return feats_h.at[idxv.at[pl.ds(p * 2 * K, 2 * K)]]

            def gdst(p):
                return gf_h.at[pl.ds((b * S + p * 2) * K, 2 * K)]

            pltpu.async_copy(gsrc(0), growv0, gsem0)

            def fpair(p, _):
                even = (p % 2) == 0
                nxt = p + 1

                @pl.when(jnp.logical_and(nxt < SP, even))
                def _():
                    pltpu.async_copy(gsrc(nxt), growv1, gsem1)

                @pl.when(jnp.logical_and(nxt < SP, jnp.logical_not(even)))
                def _():
                    pltpu.async_copy(gsrc(nxt), growv0, gsem0)

                @pl.when(even)
                def _():
                    pltpu.make_async_copy(
                        feats_h.at[pl.ds(0, 2 * K)], growv0, gsem0).wait()
                    pltpu.sync_copy(growv0, gdst(p))

                @pl.when(jnp.logical_not(even))
                def _():
                    pltpu.make_async_copy(
                        feats_h.at[pl.ds(0, 2 * K)], growv1, gsem1).wait()
                    pltpu.sync_copy(growv1, gdst(p))
                return 0

            lax.fori_loop(0, SP, fpair, 0)

    return functools.partial(
        pl.kernel, body, out_type=tuple(out_type), mesh=mesh,
        scratch_types=tuple(scratch), interpret=interpret,
        compiler_params=pltpu.CompilerParams(needs_layout_passes=False))()


def _sqdist(src, dst):
    return (jnp.sum(src ** 2, -1)[:, :, None]
            + jnp.sum(dst ** 2, -1)[:, None, :]
            - 2.0 * jnp.einsum('bsc,bnc->bsn', src, dst))


def _gather_rows(points, idx):
    return jax.vmap(lambda p, i: p[i])(points, idx)


def _fps(xyz, npoint):
    B, N, _ = xyz.shape

    def step(carry, _):
        distance, farthest = carry
        centroid = jax.vmap(lambda p, f: p[f])(xyz, farthest)[:, None, :]
        dist = jnp.sum((xyz - centroid) ** 2, -1)
        distance = jnp.minimum(distance, dist)
        new_farthest = jnp.argmax(distance, axis=-1).astype(jnp.int32)
        return (distance, new_farthest), farthest

    init = (jnp.full((B, N), 1e10, jnp.float32), jnp.zeros((B,), jnp.int32))
    _, centroids = jax.lax.scan(step, init, None, length=npoint)
    return jnp.transpose(centroids)


def _ball_query(radius, nsample, xyz, new_xyz):
    B, S, _ = new_xyz.shape
    N = xyz.shape[1]
    sqrdists = _sqdist(new_xyz, xyz)
    group_idx = jnp.broadcast_to(jnp.arange(N, dtype=jnp.int32), (B, S, N))
    group_idx = jnp.where(sqrdists > radius ** 2, N, group_idx)
    group_idx = jnp.sort(group_idx, axis=-1)[:, :, :nsample]
    group_first = jnp.broadcast_to(group_idx[:, :, 0:1], group_idx.shape)
    group_idx = jnp.where(group_idx == N, group_first, group_idx)
    return group_idx


def _conv_bn_relu(x, layer):
    W, b, gamma, beta = layer
    x = jnp.einsum('bskc,cd->bskd', x, W) + b
    mean = jnp.mean(x, axis=(0, 1, 2), keepdims=True)
    var = jnp.var(x, axis=(0, 1, 2), keepdims=True)
    x = (x - mean) / jnp.sqrt(var + 1e-5) * gamma + beta
    return jax.nn.relu(x)


def _sa(xyz, points, npoint, radius, nsample, layers, group_all):
    B = xyz.shape[0]
    if group_all:
        new_xyz = jnp.zeros((B, 1, 3), xyz.dtype)
        new_points = xyz[:, None, :, :]
        if points is not None:
            new_points = jnp.concatenate([new_points, points[:, None, :, :]], -1)
    else:
        fps_idx = _fps(xyz, npoint)
        new_xyz = _gather_rows(xyz, fps_idx)
        idx = _ball_query(radius, nsample, xyz, new_xyz)
        grouped_xyz = _gather_rows(xyz, idx) - new_xyz[:, :, None, :]
        if points is not None:
            new_points = jnp.concatenate([grouped_xyz, _gather_rows(points, idx)], -1)
        else:
            new_points = grouped_xyz
    for layer in layers:
        new_points = _conv_bn_relu(new_points, layer)
    new_points = jnp.max(new_points, axis=2)
    return new_xyz, new_points


def _mlp_pool(new_points, layers):
    for layer in layers:
        new_points = _conv_bn_relu(new_points, layer)
    return jnp.max(new_points, axis=2)


_R = 4096


def _dotT(w_ref, x_ref, contract_x):
    return lax.dot_general(
        w_ref[...], x_ref[...], (((0,), (contract_x,)), ((), ())),
        preferred_element_type=jnp.float32)


def _bn_scale_shift(acc, P, gamma, beta):
    C = acc.shape[0] // 2
    mean = jnp.sum(acc[:C], axis=1) / P
    var = jnp.sum(acc[C:], axis=1) / P - mean * mean
    s = gamma / jnp.sqrt(var + 1e-5)
    sh = beta - mean * s
    return s[:, None], sh[:, None]


def _tc_two_layer_stage(xt, gf, layers, K, P):
    (W1, b1, ga1, be1), (W2, b2, ga2, be2) = layers
    Cf = 0 if gf is None else gf.shape[1]
    C1, C2 = W1.shape[1], W2.shape[1]
    _Rl = min(_R, P)
    G = _Rl // K
    grid = P // _Rl
    W1x = jnp.zeros((8, C1), jnp.float32).at[:3].set(W1[:3])
    W1f = W1[3:] if Cf else None

    wspec = lambda shape: pl.BlockSpec(shape, lambda i: (0, 0))
    xt_spec = pl.BlockSpec((8, _Rl), lambda i: (0, i))
    gf_spec = pl.BlockSpec((_Rl, Cf), lambda i: (i, 0))
    cparams = pltpu.CompilerParams(dimension_semantics=("arbitrary",))

    def y1_of(xt_ref, gf_ref, w1x_ref, w1f_ref, b1_ref):
        y = _dotT(w1x_ref, xt_ref, 0)
        if Cf:
            y = y + _dotT(w1f_ref, gf_ref, 1)
        return y + b1_ref[...]

    gf_args = ((gf,) if Cf else ())
    gf_specs = ((gf_spec,) if Cf else ())
    w1f_specs = ((wspec((Cf, C1)),) if Cf else ())
    w1f_args = ((W1f,) if Cf else ())

    def ka(*refs):
        if Cf:
            xt_ref, gf_ref, w1x_ref, w1f_ref, b1_ref, acc_ref = refs
        else:
            xt_ref, w1x_ref, b1_ref, acc_ref = refs
            gf_ref = w1f_ref = None
        y = y1_of(xt_ref, gf_ref, w1x_ref, w1f_ref, b1_ref)

        @pl.when(pl.program_id(0) == 0)
        def _():
            acc_ref[...] = jnp.zeros_like(acc_ref)
        acc_ref[0:C1, :] += y
        acc_ref[C1:, :] += y * y

    acc1 = pl.pallas_call(
        ka, grid=(grid,),
        in_specs=[xt_spec, *gf_specs, wspec((8, C1)), *w1f_specs,
                  wspec((C1, 1))],
        out_specs=pl.BlockSpec((2 * C1, _Rl), lambda i: (0, 0)),
        out_shape=jax.ShapeDtypeStruct((2 * C1, _Rl), jnp.float32),
        compiler_params=cparams, interpret=_INTERPRET,
    )(xt, *gf_args, W1x, *w1f_args, b1[:, None])
    s1, sh1 = _bn_scale_shift(acc1, P, ga1, be1)

    def kb(*refs):
        if Cf:
            (xt_ref, gf_ref, w1x_ref, w1f_ref, b1_ref, s1_ref, sh1_ref,
             w2_ref, b2_ref, acc_ref) = refs
        else:
            (xt_ref, w1x_ref, b1_ref, s1_ref, sh1_ref,
             w2_ref, b2_ref, acc_ref) = refs
            gf_ref = w1f_ref = None
        y = y1_of(xt_ref, gf_ref, w1x_ref, w1f_ref, b1_ref)
        h1 = jnp.maximum(y * s1_ref[...] + sh1_ref[...], 0.0)
        y2 = _dotT(w2_ref, h1, 0) + b2_ref[...]

        @pl.when(pl.program_id(0) == 0)
        def _():
            acc_ref[...] = jnp.zeros_like(acc_ref)
        acc_ref[0:C2, :] += y2
        acc_ref[C2:, :] += y2 * y2

    acc2 = pl.pallas_call(
        kb, grid=(grid,),
        in_specs=[xt_spec, *gf_specs, wspec((8, C1)), *w1f_specs,
                  wspec((C1, 1)), wspec((C1, 1)), wspec((C1, 1)),
                  wspec((C1, C2)), wspec((C2, 1))],
        out_specs=pl.BlockSpec((2 * C2, _Rl), lambda i: (0, 0)),
        out_shape=jax.ShapeDtypeStruct((2 * C2, _Rl), jnp.float32),
        compiler_params=cparams, interpret=_INTERPRET,
    )(xt, *gf_args, W1x, *w1f_args, b1[:, None], s1, sh1, W2, b2[:, None])
    s2, sh2 = _bn_scale_shift(acc2, P, ga2, be2)

    def kc(*refs):
        if Cf:
            (xt_ref, gf_ref, w1x_ref, w1f_ref, b1_ref, s1_ref, sh1_ref,
             w2_ref, b2_ref, s2_ref, sh2_ref, out_ref) = refs
        else:
            (xt_ref, w1x_ref, b1_ref, s1_ref, sh1_ref,
             w2_ref, b2_ref, s2_ref, sh2_ref, out_ref) = refs
            gf_ref = w1f_ref = None
        y = y1_of(xt_ref, gf_ref, w1x_ref, w1f_ref, b1_ref)
        h1 = jnp.maximum(y * s1_ref[...] + sh1_ref[...], 0.0)
        y2 = _dotT(w2_ref, h1, 0) + b2_ref[...]
        h2 = jnp.maximum(y2 * s2_ref[...] + sh2_ref[...], 0.0)
        out_ref[...] = jnp.concatenate(
            [jnp.max(h2[:, g * K:(g + 1) * K], axis=1, keepdims=True)
             for g in range(G)], axis=1)[None]

    pooled = pl.pallas_call(
        kc, grid=(grid,),
        in_specs=[xt_spec, *gf_specs, wspec((8, C1)), *w1f_specs,
                  wspec((C1, 1)), wspec((C1, 1)), wspec((C1, 1)),
                  wspec((C1, C2)), wspec((C2, 1)),
                  wspec((C2, 1)), wspec((C2, 1))],
        out_specs=pl.BlockSpec((1, C2, G), lambda i: (i, 0, 0)),
        out_shape=jax.ShapeDtypeStruct((grid, C2, G), jnp.float32),
        compiler_params=cparams, interpret=_INTERPRET,
    )(xt, *gf_args, W1x, *w1f_args, b1[:, None], s1, sh1, W2, b2[:, None],
      s2, sh2)
    return jnp.transpose(pooled, (0, 2, 1)).reshape(P // K, C2)


def _tc_sa3_stage(xt, feats, layer, K):
    W, b, ga, be = layer
    P = xt.shape[1]
    C2 = W.shape[1]
    G = P // K
    Wx = jnp.zeros((8, C2), jnp.float32).at[:3].set(W[:3])
    Wf = W[3:]
    Cf = Wf.shape[0]

    def body(xt_ref, f_ref, wx_ref, wf_ref, b_ref, ga_ref, be_ref, out_ref):
        y = _dotT(wx_ref, xt_ref, 0) + _dotT(wf_ref, f_ref, 1) + b_ref[...]
        mean = jnp.sum(y, axis=1, keepdims=True) / P
        var = jnp.sum(y * y, axis=1, keepdims=True) / P - mean * mean
        s = ga_ref[...] / jnp.sqrt(var + 1e-5)
        sh = be_ref[...] - mean * s
        h = jnp.maximum(y * s + sh, 0.0)
        out_ref[...] = jnp.concatenate(
            [jnp.max(h[:, g * K:(g + 1) * K], axis=1, keepdims=True)
             for g in range(G)], axis=1)

    return pl.pallas_call(
        body,
        out_shape=jax.ShapeDtypeStruct((C2, G), jnp.float32),
        interpret=_INTERPRET,
    )(xt, feats, Wx, Wf, b[:, None], ga[:, None], be[:, None])


def _pad8_planes(planes, P):
    xt = jnp.zeros((8, P), jnp.float32)
    for i, p in enumerate(planes):
        xt = xt.at[i].set(p.reshape(P))
    return xt


_INTERPRET = False


def kernel(xyz, params):
    B = xyz.shape[0]
    px, py, pz = xyz[:, 0, :], xyz[:, 1, :], xyz[:, 2, :]

    fps1 = _make_sc_fps_kernel(1024, 512, interpret=_INTERPRET)
    nx1, ny1, nz1 = fps1(px, py, pz)
    new_xyz1 = jnp.stack([nx1, ny1, nz1], axis=-1)
    xyz_t = jnp.transpose(xyz, (0, 2, 1))
    sq1 = _sqdist(new_xyz1, xyz_t)
    sc1 = _make_sc_group_kernel(1024, 512, 32, 0.2, 0, interpret=_INTERPRET)
    g1x, g1y, g1z = sc1(px, py, pz, nx1, ny1, nz1, sq1)
    P1 = B * 512 * 32
    xt1 = _pad8_planes([g1x, g1y, g1z], P1)
    l1_points = _tc_two_layer_stage(xt1, None, params['sa1'], 32, P1)

    fps2 = _make_sc_fps_kernel(512, 128, interpret=_INTERPRET)
    nx2, ny2, nz2 = fps2(nx1, ny1, nz1)
    new_xyz2 = jnp.stack([nx2, ny2, nz2], axis=-1)
    sq2 = _sqdist(new_xyz2, new_xyz1)
    sc2 = _make_sc_group_kernel(512, 128, 64, 0.4, 128, interpret=_INTERPRET)
    g2x, g2y, g2z, g2f = sc2(nx1, ny1, nz1, nx2, ny2, nz2, sq2, l1_points)
    P2 = B * 128 * 64
    xt2 = _pad8_planes([g2x, g2y, g2z], P2)
    l2_points = _tc_two_layer_stage(xt2, g2f, params['sa2'], 64, P2)

    xt3 = _pad8_planes([nx2, ny2, nz2], B * 128)
    pooled3 = _tc_sa3_stage(xt3, l2_points, params['sa3'][0], 128)

    x = pooled3.T
    l3 = x[:, :, None]
    return x, l3
